# baseline (speedup 1.0000x reference)
import functools

import jax
import jax.numpy as jnp
from jax import lax
from jax.experimental import pallas as pl
from jax.experimental.pallas import tpu as pltpu
from jax.experimental.pallas import tpu_sc as plsc

N = 10000
NP = 10240
E = 160000
NC = 2
NS = 16
NW = NC * NS
EB = 40
EBD = 40
ERows = E // EB
RPT = NP // NS

BM = 640

_MESH = plsc.VectorSubcoreMesh(core_axis_name="c", subcore_axis_name="s")


def _deg_body(dst_hbm, zeros16_hbm, out_hbm, dst_v, ones_v, acc, sem):
    cid = lax.axis_index("c")
    sid = lax.axis_index("s")
    wid = sid * NC + cid
    pltpu.sync_copy(dst_hbm.at[wid], dst_v)
    for i in range(EBD):
        ones_v[i, :] = jnp.ones((16,), jnp.float32)
    r0 = pl.multiple_of(sid * RPT, RPT)
    pltpu.sync_copy(zeros16_hbm, acc.at[pl.ds(r0, RPT)])
    plsc.subcore_barrier()

    def body(j, carry):
        pltpu.sync_copy(ones_v, acc.at[dst_v.at[j]], add=True)
        return carry

    lax.fori_loop(0, E // EBD // NW, body, 0)
    plsc.subcore_barrier()
    pltpu.sync_copy(acc.at[pl.ds(r0, RPT)], out_hbm.at[cid].at[pl.ds(r0, RPT)])


_deg_call = functools.partial(
    pl.kernel,
    mesh=_MESH,
    out_type=jax.ShapeDtypeStruct((NC, NP, 16), jnp.float32),
    scratch_types=[
        pltpu.VMEM((E // EBD // NW, EBD), jnp.int32),
        pltpu.VMEM((EBD, 16), jnp.float32),
        pltpu.VMEM_SHARED((NP, 16), jnp.float32),
        pltpu.SemaphoreType.DMA,
    ],
)(_deg_body)


NBB = 25
NBLK = ERows // NS // NBB


def _make_agg(D):
    cpc = D // 128 // NC

    def body(y_hbm, src_hbm, dst_hbm, out_hbm, src_v, dst_v, rows_v, acc,
             gsem, ssem, isem):
        cid = lax.axis_index("c")
        sid = lax.axis_index("s")
        r0 = pl.multiple_of(sid * RPT, RPT)

        def drain_scatter(slot):
            pltpu.make_async_copy(rows_v.at[0], acc.at[pl.ds(r0, EB)],
                                  ssem.at[slot]).wait()

        def drain_gather(slot):
            pltpu.make_async_copy(y_hbm.at[pl.ds(0, EB), pl.ds(0, 128)],
                                  rows_v.at[0], gsem.at[slot]).wait()

        def fetch_idx(blk, slot):
            pltpu.async_copy(src_hbm.at[sid].at[blk], src_v.at[slot], isem)
            pltpu.async_copy(dst_hbm.at[sid].at[blk], dst_v.at[slot], isem)

        def drain_idx():
            pltpu.make_async_copy(src_hbm.at[sid].at[0], src_v.at[0],
                                  isem).wait()
            pltpu.make_async_copy(dst_hbm.at[sid].at[0], dst_v.at[0],
                                  isem).wait()

        for cc in range(cpc):
            col = pl.multiple_of((cid * cpc + cc) * 128, 128)
            for i in range(EB):
                for jj in range(8):
                    rows_v[0, i, pl.ds(jj * 16, 16)] = jnp.zeros(
                        (16,), jnp.float32)
            for k in range(RPT // EB):
                pltpu.sync_copy(rows_v.at[0], acc.at[pl.ds(r0 + k * EB, EB)])
            fetch_idx(0, 0)
            plsc.subcore_barrier()

            def blk_body(blk, carry):
                slot = lax.rem(blk, 2)
                drain_idx()
                @pl.when(blk + 1 < NBLK)
                def _():
                    fetch_idx(blk + 1, 1 - slot)
                sv = src_v.at[slot]
                dv = dst_v.at[slot]
                pltpu.async_copy(y_hbm.at[sv.at[0], pl.ds(col, 128)],
                                 rows_v.at[0], gsem.at[0])
                pltpu.async_copy(y_hbm.at[sv.at[1], pl.ds(col, 128)],
                                 rows_v.at[1], gsem.at[1])

                def ebody(j, carry2):
                    buf = lax.rem(j, 4)
                    nbuf = lax.rem(j + 2, 4)

                    @pl.when(j >= 2)
                    def _():
                        drain_scatter(nbuf)

                    @pl.when(j + 2 < NBB)
                    def _():
                        pltpu.async_copy(
                            y_hbm.at[sv.at[j + 2], pl.ds(col, 128)],
                            rows_v.at[nbuf], gsem.at[nbuf])

                    drain_gather(buf)
                    pltpu.async_copy(rows_v.at[buf],
                                     acc.at[dv.at[j]], ssem.at[buf],
                                     add=True)
                    return carry2

                lax.fori_loop(0, NBB, ebody, 0)
                drain_scatter((NBB - 2) % 4)
                drain_scatter((NBB - 1) % 4)
                return carry

            lax.fori_loop(0, NBLK, blk_body, 0)
            plsc.subcore_barrier()
            pltpu.sync_copy(acc.at[pl.ds(r0, RPT)],
                            out_hbm.at[pl.ds(r0, RPT), pl.ds(col, 128)])
            plsc.subcore_barrier()

    return functools.partial(
        pl.kernel,
        mesh=_MESH,
        out_type=jax.ShapeDtypeStruct((NP, D), jnp.float32),
        scratch_types=[
            pltpu.VMEM((2, NBB, EB), jnp.int32),
            pltpu.VMEM((2, NBB, EB), jnp.int32),
            pltpu.VMEM((4, EB, 128), jnp.float32),
            pltpu.VMEM_SHARED((NP, 128), jnp.float32),
            pltpu.SemaphoreType.DMA((4,)),
            pltpu.SemaphoreType.DMA((4,)),
            pltpu.SemaphoreType.DMA,
        ],
    )(body)


_agg512 = _make_agg(512)
_agg256 = _make_agg(256)


def _dinv_of(degp_ref):
    deg = degp_ref[0, :, 0:1] + degp_ref[1, :, 0:1] + 1.0
    return lax.rsqrt(deg)


def _mm_first_body(x_ref, w_ref, degp_ref, y_ref):
    dinv = _dinv_of(degp_ref)
    y_ref[...] = dinv * jnp.dot(x_ref[...], w_ref[...],
                                preferred_element_type=jnp.float32)


def _mm_mid_body(agg_ref, y_ref, degp_ref, b_ref, w_ref, out_ref):
    dinv = _dinv_of(degp_ref)
    h = jnp.maximum(dinv * (agg_ref[...] + y_ref[...]) + b_ref[...], 0.0)
    out_ref[...] = dinv * jnp.dot(h, w_ref[...],
                                  preferred_element_type=jnp.float32)


def _final_body(agg_ref, y_ref, degp_ref, b_ref, out_ref):
    dinv = _dinv_of(degp_ref)
    out_ref[...] = dinv * (agg_ref[...] + y_ref[...]) + b_ref[...]


def _degp_spec():
    return pl.BlockSpec((2, BM, 16), lambda i: (0, i, 0))


def _mm_first(x, w, degp):
    kin, kout = w.shape
    return pl.pallas_call(
        _mm_first_body,
        grid=(NP // BM,),
        in_specs=[
            pl.BlockSpec((BM, kin), lambda i: (i, 0)),
            pl.BlockSpec((kin, kout), lambda i: (0, 0)),
            _degp_spec(),
        ],
        out_specs=pl.BlockSpec((BM, kout), lambda i: (i, 0)),
        out_shape=jax.ShapeDtypeStruct((NP, kout), jnp.float32),
    )(x, w, degp)


def _mm_mid(agg, y, degp, b, w):
    kin, kout = w.shape
    return pl.pallas_call(
        _mm_mid_body,
        grid=(NP // BM,),
        in_specs=[
            pl.BlockSpec((BM, kin), lambda i: (i, 0)),
            pl.BlockSpec((BM, kin), lambda i: (i, 0)),
            _degp_spec(),
            pl.BlockSpec((1, kin), lambda i: (0, 0)),
            pl.BlockSpec((kin, kout), lambda i: (0, 0)),
        ],
        out_specs=pl.BlockSpec((BM, kout), lambda i: (i, 0)),
        out_shape=jax.ShapeDtypeStruct((NP, kout), jnp.float32),
    )(agg, y, degp, b, w)


def _final(agg, y, degp, b):
    kout = agg.shape[1]
    return pl.pallas_call(
        _final_body,
        grid=(NP // BM,),
        in_specs=[
            pl.BlockSpec((BM, kout), lambda i: (i, 0)),
            pl.BlockSpec((BM, kout), lambda i: (i, 0)),
            _degp_spec(),
            pl.BlockSpec((1, kout), lambda i: (0, 0)),
        ],
        out_specs=pl.BlockSpec((BM, kout), lambda i: (i, 0)),
        out_shape=jax.ShapeDtypeStruct((NP, kout), jnp.float32),
    )(agg, y, degp, b)


def kernel(x, edge_index, W1, b1, W2, b2, W3, b3):
    src = edge_index[0].astype(jnp.int32).reshape(NS, NBLK, NBB, EB)
    dst = edge_index[1].astype(jnp.int32)
    dst_deg = dst.reshape(NW, E // EBD // NW, EBD)
    dst_agg = dst.reshape(NS, NBLK, NBB, EB)
    zeros16 = jnp.zeros((RPT, 16), jnp.float32)
    x_p = jnp.pad(x, ((0, NP - N), (0, 0)))

    degp = _deg_call(dst_deg, zeros16)

    y1 = _mm_first(x_p, W1, degp)
    agg1 = _agg512(y1, src, dst_agg)
    y2 = _mm_mid(agg1, y1, degp, b1.reshape(1, -1), W2)
    agg2 = _agg512(y2, src, dst_agg)
    y3 = _mm_mid(agg2, y2, degp, b2.reshape(1, -1), W3)
    agg3 = _agg256(y3, src, dst_agg)
    return _final(agg3, y3, degp, b3.reshape(1, -1))[:N]


# speedup vs baseline: 1.0643x; 1.0643x over previous
import functools

import jax
import jax.numpy as jnp
from jax import lax
from jax.experimental import pallas as pl
from jax.experimental.pallas import tpu as pltpu
from jax.experimental.pallas import tpu_sc as plsc

N = 10000
NP = 10240
E = 160000
NC = 2
NS = 16
NW = NC * NS
EB = 80
EBD = 40
ERows = E // EB
RPT = NP // NS

BM = 640

_MESH = plsc.VectorSubcoreMesh(core_axis_name="c", subcore_axis_name="s")


def _deg_body(dst_hbm, zeros16_hbm, out_hbm, dst_v, ones_v, acc, sem):
    cid = lax.axis_index("c")
    sid = lax.axis_index("s")
    wid = sid * NC + cid
    pltpu.sync_copy(dst_hbm.at[wid], dst_v)
    for i in range(EBD):
        ones_v[i, :] = jnp.ones((16,), jnp.float32)
    r0 = pl.multiple_of(sid * RPT, RPT)
    pltpu.sync_copy(zeros16_hbm, acc.at[pl.ds(r0, RPT)])
    plsc.subcore_barrier()

    def body(j, carry):
        pltpu.sync_copy(ones_v, acc.at[dst_v.at[j]], add=True)
        return carry

    lax.fori_loop(0, E // EBD // NW, body, 0)
    plsc.subcore_barrier()
    pltpu.sync_copy(acc.at[pl.ds(r0, RPT)], out_hbm.at[cid].at[pl.ds(r0, RPT)])


_deg_call = functools.partial(
    pl.kernel,
    mesh=_MESH,
    out_type=jax.ShapeDtypeStruct((NC, NP, 16), jnp.float32),
    scratch_types=[
        pltpu.VMEM((E // EBD // NW, EBD), jnp.int32),
        pltpu.VMEM((EBD, 16), jnp.float32),
        pltpu.VMEM_SHARED((NP, 16), jnp.float32),
        pltpu.SemaphoreType.DMA,
    ],
)(_deg_body)


NBB = 25
NBLK = ERows // NS // NBB


def _make_agg(D):
    cpc = D // 128 // NC

    def body(y_hbm, src_hbm, dst_hbm, out_hbm, src_v, dst_v, rows_v, acc,
             gsem, ssem, isem):
        cid = lax.axis_index("c")
        sid = lax.axis_index("s")
        r0 = pl.multiple_of(sid * RPT, RPT)

        def drain_scatter(slot):
            pltpu.make_async_copy(rows_v.at[0], acc.at[pl.ds(r0, EB)],
                                  ssem.at[slot]).wait()

        def drain_gather(slot):
            pltpu.make_async_copy(y_hbm.at[pl.ds(0, EB), pl.ds(0, 128)],
                                  rows_v.at[0], gsem.at[slot]).wait()

        def fetch_idx(blk, slot):
            pltpu.async_copy(src_hbm.at[sid].at[blk], src_v.at[slot], isem)
            pltpu.async_copy(dst_hbm.at[sid].at[blk], dst_v.at[slot], isem)

        def drain_idx():
            pltpu.make_async_copy(src_hbm.at[sid].at[0], src_v.at[0],
                                  isem).wait()
            pltpu.make_async_copy(dst_hbm.at[sid].at[0], dst_v.at[0],
                                  isem).wait()

        for cc in range(cpc):
            col = pl.multiple_of((cid * cpc + cc) * 128, 128)
            for i in range(EB):
                for jj in range(8):
                    rows_v[0, i, pl.ds(jj * 16, 16)] = jnp.zeros(
                        (16,), jnp.float32)
            for k in range(RPT // EB):
                pltpu.sync_copy(rows_v.at[0], acc.at[pl.ds(r0 + k * EB, EB)])
            fetch_idx(0, 0)
            plsc.subcore_barrier()

            def blk_body(blk, carry):
                slot = lax.rem(blk, 2)
                drain_idx()
                @pl.when(blk + 1 < NBLK)
                def _():
                    fetch_idx(blk + 1, 1 - slot)
                sv = src_v.at[slot]
                dv = dst_v.at[slot]
                pltpu.async_copy(y_hbm.at[sv.at[0], pl.ds(col, 128)],
                                 rows_v.at[0], gsem.at[0])

                def ebody(j, carry2):
                    buf = lax.rem(j, 3)
                    nbuf = lax.rem(j + 1, 3)

                    @pl.when(j >= 2)
                    def _():
                        drain_scatter(nbuf)

                    @pl.when(j + 1 < NBB)
                    def _():
                        pltpu.async_copy(
                            y_hbm.at[sv.at[j + 1], pl.ds(col, 128)],
                            rows_v.at[nbuf], gsem.at[nbuf])

                    drain_gather(buf)
                    pltpu.async_copy(rows_v.at[buf],
                                     acc.at[dv.at[j]], ssem.at[buf],
                                     add=True)
                    return carry2

                lax.fori_loop(0, NBB, ebody, 0)
                drain_scatter((NBB - 2) % 3)
                drain_scatter((NBB - 1) % 3)
                return carry

            lax.fori_loop(0, NBLK, blk_body, 0)
            plsc.subcore_barrier()
            pltpu.sync_copy(acc.at[pl.ds(r0, RPT)],
                            out_hbm.at[pl.ds(r0, RPT), pl.ds(col, 128)])
            plsc.subcore_barrier()

    return functools.partial(
        pl.kernel,
        mesh=_MESH,
        out_type=jax.ShapeDtypeStruct((NP, D), jnp.float32),
        scratch_types=[
            pltpu.VMEM((2, NBB, EB), jnp.int32),
            pltpu.VMEM((2, NBB, EB), jnp.int32),
            pltpu.VMEM((3, EB, 128), jnp.float32),
            pltpu.VMEM_SHARED((NP, 128), jnp.float32),
            pltpu.SemaphoreType.DMA((3,)),
            pltpu.SemaphoreType.DMA((3,)),
            pltpu.SemaphoreType.DMA,
        ],
    )(body)


_agg512 = _make_agg(512)
_agg256 = _make_agg(256)


def _dinv_of(degp_ref):
    deg = degp_ref[0, :, 0:1] + degp_ref[1, :, 0:1] + 1.0
    return lax.rsqrt(deg)


def _mm_first_body(x_ref, w_ref, degp_ref, y_ref):
    dinv = _dinv_of(degp_ref)
    y_ref[...] = dinv * jnp.dot(x_ref[...], w_ref[...],
                                preferred_element_type=jnp.float32)


def _mm_mid_body(agg_ref, y_ref, degp_ref, b_ref, w_ref, out_ref):
    dinv = _dinv_of(degp_ref)
    h = jnp.maximum(dinv * (agg_ref[...] + y_ref[...]) + b_ref[...], 0.0)
    out_ref[...] = dinv * jnp.dot(h, w_ref[...],
                                  preferred_element_type=jnp.float32)


def _final_body(agg_ref, y_ref, degp_ref, b_ref, out_ref):
    dinv = _dinv_of(degp_ref)
    out_ref[...] = dinv * (agg_ref[...] + y_ref[...]) + b_ref[...]


def _degp_spec():
    return pl.BlockSpec((2, BM, 16), lambda i: (0, i, 0))


def _mm_first(x, w, degp):
    kin, kout = w.shape
    return pl.pallas_call(
        _mm_first_body,
        grid=(NP // BM,),
        in_specs=[
            pl.BlockSpec((BM, kin), lambda i: (i, 0)),
            pl.BlockSpec((kin, kout), lambda i: (0, 0)),
            _degp_spec(),
        ],
        out_specs=pl.BlockSpec((BM, kout), lambda i: (i, 0)),
        out_shape=jax.ShapeDtypeStruct((NP, kout), jnp.float32),
    )(x, w, degp)


def _mm_mid(agg, y, degp, b, w):
    kin, kout = w.shape
    return pl.pallas_call(
        _mm_mid_body,
        grid=(NP // BM,),
        in_specs=[
            pl.BlockSpec((BM, kin), lambda i: (i, 0)),
            pl.BlockSpec((BM, kin), lambda i: (i, 0)),
            _degp_spec(),
            pl.BlockSpec((1, kin), lambda i: (0, 0)),
            pl.BlockSpec((kin, kout), lambda i: (0, 0)),
        ],
        out_specs=pl.BlockSpec((BM, kout), lambda i: (i, 0)),
        out_shape=jax.ShapeDtypeStruct((NP, kout), jnp.float32),
    )(agg, y, degp, b, w)


def _final(agg, y, degp, b):
    kout = agg.shape[1]
    return pl.pallas_call(
        _final_body,
        grid=(NP // BM,),
        in_specs=[
            pl.BlockSpec((BM, kout), lambda i: (i, 0)),
            pl.BlockSpec((BM, kout), lambda i: (i, 0)),
            _degp_spec(),
            pl.BlockSpec((1, kout), lambda i: (0, 0)),
        ],
        out_specs=pl.BlockSpec((BM, kout), lambda i: (i, 0)),
        out_shape=jax.ShapeDtypeStruct((NP, kout), jnp.float32),
    )(agg, y, degp, b)


def kernel(x, edge_index, W1, b1, W2, b2, W3, b3):
    src = edge_index[0].astype(jnp.int32).reshape(NS, NBLK, NBB, EB)
    dst = edge_index[1].astype(jnp.int32)
    dst_deg = dst.reshape(NW, E // EBD // NW, EBD)
    dst_agg = dst.reshape(NS, NBLK, NBB, EB)
    zeros16 = jnp.zeros((RPT, 16), jnp.float32)
    x_p = jnp.pad(x, ((0, NP - N), (0, 0)))

    degp = _deg_call(dst_deg, zeros16)

    y1 = _mm_first(x_p, W1, degp)
    agg1 = _agg512(y1, src, dst_agg)
    y2 = _mm_mid(agg1, y1, degp, b1.reshape(1, -1), W2)
    agg2 = _agg512(y2, src, dst_agg)
    y3 = _mm_mid(agg2, y2, degp, b2.reshape(1, -1), W3)
    agg3 = _agg256(y3, src, dst_agg)
    return _final(agg3, y3, degp, b3.reshape(1, -1))[:N]
